# baseline (speedup 1.0000x reference)
import numpy as np
import jax
import jax.numpy as jnp
from jax.experimental import pallas as pl
from jax.experimental.pallas import tpu as pltpu

DIM = 64
BASE = 10000.0
LANES = 128

_W = np.power(BASE, -np.arange(32) / 32.0)


def _tile(vec64):
    return np.broadcast_to(vec64.astype(np.float32)[:, None], (32, LANES))


def _consts_for(block_cols: int) -> np.ndarray:
    return np.concatenate(
        [
            _tile(_W),
            _tile(np.cos(128.0 * _W)),
            _tile(np.sin(128.0 * _W)),
            _tile(np.cos(float(block_cols) * _W)),
            _tile(np.sin(float(block_cols) * _W)),
        ],
        axis=0,
    )


def _rope_kernel(const_ref, cos_ref, sin_ref, uc_ref, us_ref):
    cols = cos_ref.shape[1]
    i = pl.program_id(0)
    w = const_ref[0:32, :]

    @pl.when(i == 0)
    def _seed():
        lane = jax.lax.broadcasted_iota(jnp.int32, (32, LANES), 1)
        phase = lane.astype(jnp.float32) * w
        uc_ref[...] = jnp.cos(phase)
        us_ref[...] = jnp.sin(phase)

    @pl.when(i > 0)
    def _advance():
        bc = const_ref[96:128, :]
        bs = const_ref[128:160, :]
        c, s = uc_ref[...], us_ref[...]
        uc_ref[...] = c * bc - s * bs
        us_ref[...] = s * bc + c * bs

    rc = const_ref[32:64, :]
    rs = const_ref[64:96, :]
    c_chunks = [uc_ref[...]]
    s_chunks = [us_ref[...]]
    for _ in range(cols // LANES - 1):
        c, s = c_chunks[-1], s_chunks[-1]
        c_chunks.append(c * rc - s * rs)
        s_chunks.append(s * rc + c * rs)
    u_c = jnp.concatenate(c_chunks, axis=1)
    u_s = jnp.concatenate(s_chunks, axis=1)
    full_c = jnp.concatenate([u_c, u_c], axis=0)
    full_s = jnp.concatenate([u_s, u_s], axis=0)

    cos_ref[0:64, :] = full_c
    cos_ref[64:128, :] = full_c
    cos_ref[128:192, :] = jnp.ones((64, cols), jnp.float32)
    sin_ref[0:64, :] = full_s
    sin_ref[64:128, :] = full_s
    sin_ref[128:192, :] = jnp.zeros((64, cols), jnp.float32)


def kernel(x):
    seq_len = x.shape[1]
    grid = 4 if seq_len % (4 * LANES) == 0 else 1
    block_cols = seq_len // grid
    consts = jnp.asarray(_consts_for(block_cols))
    cos_p, sin_p = pl.pallas_call(
        _rope_kernel,
        grid=(grid,),
        in_specs=[pl.BlockSpec((160, LANES), lambda i: (0, 0))],
        out_specs=[
            pl.BlockSpec((192, block_cols), lambda i: (0, i)),
            pl.BlockSpec((192, block_cols), lambda i: (0, i)),
        ],
        out_shape=[
            jax.ShapeDtypeStruct((192, seq_len), jnp.float32),
            jax.ShapeDtypeStruct((192, seq_len), jnp.float32),
        ],
        scratch_shapes=[
            pltpu.VMEM((32, LANES), jnp.float32),
            pltpu.VMEM((32, LANES), jnp.float32),
        ],
    )(consts)
    shape = (1, seq_len, 3, 1, DIM)
    cos = cos_p.reshape(3, DIM, seq_len).transpose(2, 0, 1).reshape(shape)
    sin = sin_p.reshape(3, DIM, seq_len).transpose(2, 0, 1).reshape(shape)
    return cos, sin
